# baseline (speedup 1.0000x reference)
import functools

import jax
import jax.numpy as jnp
from jax import lax
from jax.experimental import pallas as pl
from jax.experimental.pallas import tpu as pltpu
from jax.experimental.pallas import tpu_sc as plsc

_T, _B, _C = 576, 64, 768
_RATIO = 0.75
_REMAIN = int(_T * (1 - _RATIO))
_MASKED = _T - _REMAIN
_NC, _NS = 2, 16
_NW = _NC * _NS
_CHUNK = 48
_MC = (_MASKED * _B) // (_NW * _CHUNK)
_DEPTH = 3
_BG = 8
_ST = 64


def _indexes():
    keys = jax.random.split(jax.random.key(42), _B)
    fwd = jax.vmap(lambda k: jax.random.permutation(k, _T))(keys).T
    bwd = jnp.argsort(fwd, axis=0)
    col = jnp.arange(_B, dtype=jnp.int32)[None, :]
    idx_m = (fwd[_REMAIN:].astype(jnp.int32) * _B + col).reshape(_NW, _MC, _CHUNK)
    fwd_p3 = fwd[:_REMAIN].astype(jnp.int32).T.reshape(_B // _BG, _BG, _REMAIN)
    return fwd, bwd, idx_m, fwd_p3


def _sc_gather_mask(mask_f, idx_m):
    mesh = plsc.VectorSubcoreMesh(core_axis_name="c", subcore_axis_name="s")

    @functools.partial(
        pl.kernel,
        mesh=mesh,
        out_type=jax.ShapeDtypeStruct((_MASKED * _B, _C), jnp.float32),
        scratch_types=(
            [pltpu.VMEM((_MC, _CHUNK), jnp.int32)]
            + [pltpu.VMEM((_CHUNK, _C), jnp.float32) for _ in range(_DEPTH)]
            + [pltpu.SemaphoreType.DMA for _ in range(2 * _DEPTH)]
        ),
    )
    def k(m_hbm, im_hbm, om_hbm, imv, *scratch):
        bufs = list(scratch[:_DEPTH])
        gsems = list(scratch[_DEPTH : 2 * _DEPTH])
        osems = list(scratch[2 * _DEPTH :])
        wid = lax.axis_index("s") * _NC + lax.axis_index("c")
        pltpu.sync_copy(im_hbm.at[wid], imv)
        n = _MC
        gh = [None] * n
        oh = [None] * n

        def out_copy(j):
            return pltpu.async_copy(
                bufs[j % _DEPTH],
                om_hbm.at[pl.ds((wid * _MC + j) * _CHUNK, _CHUNK)],
                osems[j % _DEPTH],
            )

        for i in range(n):
            if i >= _DEPTH:
                oh[i - _DEPTH].wait()
            gh[i] = pltpu.async_copy(m_hbm.at[imv.at[i]], bufs[i % _DEPTH], gsems[i % _DEPTH])
            if i >= 1:
                gh[i - 1].wait()
                oh[i - 1] = out_copy(i - 1)
        gh[n - 1].wait()
        oh[n - 1] = out_copy(n - 1)
        for j in range(max(0, n - _DEPTH), n):
            oh[j].wait()

    return k(mask_f, idx_m)


def _tc_onehot_patches(patches, fwd_p):
    n_bg = _B // _BG
    n_st = _T // _ST

    def body(f_ref, x_ref, o_ref):
        st = pl.program_id(1)

        @pl.when(st == 0)
        def _init():
            o_ref[...] = jnp.zeros_like(o_ref)

        s_base = st * _ST
        iota_s = lax.broadcasted_iota(jnp.int32, (_ST, _REMAIN), 0) + s_base
        for j in range(_BG):
            f_j = f_ref[0, j, :]
            oh = (iota_s == f_j[None, :]).astype(jnp.float32)
            x_j = x_ref[:, j, :]
            o_ref[:, j, :] += lax.dot_general(
                oh, x_j, (((0,), (0,)), ((), ())),
                precision=jax.lax.Precision.HIGHEST,
                preferred_element_type=jnp.float32,
            )

    return pl.pallas_call(
        body,
        grid=(n_bg, n_st),
        in_specs=[
            pl.BlockSpec((1, _BG, _REMAIN), lambda bg, st: (bg, 0, 0)),
            pl.BlockSpec((_ST, _BG, _C), lambda bg, st: (st, bg, 0)),
        ],
        out_specs=pl.BlockSpec((_REMAIN, _BG, _C), lambda bg, st: (0, bg, 0)),
        out_shape=jax.ShapeDtypeStruct((_REMAIN, _B, _C), jnp.float32),
        compiler_params=pltpu.CompilerParams(
            dimension_semantics=("parallel", "arbitrary"),
        ),
    )(fwd_p, patches)


def kernel(patches, mask_patches):
    fwd, bwd, idx_m, fwd_p = _indexes()
    out_m = _sc_gather_mask(mask_patches.reshape(_T * _B, _C), idx_m)
    out_p = _tc_onehot_patches(patches, fwd_p)
    return (
        out_p,
        out_m.reshape(_MASKED, _B, _C),
        fwd,
        bwd,
    )


# speedup vs baseline: 1.5350x; 1.5350x over previous
import functools

import jax
import jax.numpy as jnp
from jax import lax
from jax.experimental import pallas as pl
from jax.experimental.pallas import tpu as pltpu
from jax.experimental.pallas import tpu_sc as plsc

_T, _B, _C = 576, 64, 768
_RATIO = 0.75
_REMAIN = int(_T * (1 - _RATIO))
_MASKED = _T - _REMAIN
_NC, _NS = 2, 16
_NW = _NC * _NS
_CHUNK = 48
_MC = (_MASKED * _B) // (_NW * _CHUNK)
_DEPTH = 3
_BG = 8
_ST = 64


def _indexes():
    keys = jax.random.split(jax.random.key(42), _B)
    fwd = jax.vmap(lambda k: jax.random.permutation(k, _T))(keys).T
    bwd = jnp.argsort(fwd, axis=0)
    col = jnp.arange(_B, dtype=jnp.int32)[None, :]
    idx_m = (fwd[_REMAIN:].astype(jnp.int32) * _B + col).reshape(_NW, _MC, _CHUNK)
    fwd_p3 = fwd[:_REMAIN].astype(jnp.int32).T.reshape(_B // _BG, _BG, _REMAIN)
    return fwd, bwd, idx_m, fwd_p3


def _sc_gather_mask(mask_f, idx_m):
    mesh = plsc.VectorSubcoreMesh(core_axis_name="c", subcore_axis_name="s")

    @functools.partial(
        pl.kernel,
        mesh=mesh,
        out_type=jax.ShapeDtypeStruct((_MASKED * _B, _C), jnp.float32),
        scratch_types=(
            [pltpu.VMEM((_MC, _CHUNK), jnp.int32)]
            + [pltpu.VMEM((_CHUNK, _C), jnp.float32) for _ in range(_DEPTH)]
            + [pltpu.SemaphoreType.DMA for _ in range(2 * _DEPTH)]
        ),
    )
    def k(m_hbm, im_hbm, om_hbm, imv, *scratch):
        bufs = list(scratch[:_DEPTH])
        gsems = list(scratch[_DEPTH : 2 * _DEPTH])
        osems = list(scratch[2 * _DEPTH :])
        wid = lax.axis_index("s") * _NC + lax.axis_index("c")
        pltpu.sync_copy(im_hbm.at[wid], imv)
        n = _MC
        gh = [None] * n
        oh = [None] * n

        def out_copy(j):
            return pltpu.async_copy(
                bufs[j % _DEPTH],
                om_hbm.at[pl.ds((wid * _MC + j) * _CHUNK, _CHUNK)],
                osems[j % _DEPTH],
            )

        for i in range(n):
            if i >= _DEPTH:
                oh[i - _DEPTH].wait()
            gh[i] = pltpu.async_copy(m_hbm.at[imv.at[i]], bufs[i % _DEPTH], gsems[i % _DEPTH])
            if i >= 1:
                gh[i - 1].wait()
                oh[i - 1] = out_copy(i - 1)
        gh[n - 1].wait()
        oh[n - 1] = out_copy(n - 1)
        for j in range(max(0, n - _DEPTH), n):
            oh[j].wait()

    return k(mask_f, idx_m)


def _tc_onehot_patches(patches, fwd_p):
    n_bg = _B // _BG
    n_st = _T // _ST

    def body(f_ref, x_ref, o_ref):
        st = pl.program_id(1)

        @pl.when(st == 0)
        def _init():
            o_ref[...] = jnp.zeros_like(o_ref)

        s_base = st * _ST
        iota_s = lax.broadcasted_iota(jnp.int32, (_ST, _REMAIN), 0) + s_base
        for j in range(_BG):
            f_j = f_ref[0, j, :]
            oh = (iota_s == f_j[None, :]).astype(jnp.float32)
            x_j = x_ref[:, j, :]
            o_ref[:, j, :] += lax.dot_general(
                oh, x_j, (((0,), (0,)), ((), ())),
                precision=jax.lax.Precision.DEFAULT,
                preferred_element_type=jnp.float32,
            )

    return pl.pallas_call(
        body,
        grid=(n_bg, n_st),
        in_specs=[
            pl.BlockSpec((1, _BG, _REMAIN), lambda bg, st: (bg, 0, 0)),
            pl.BlockSpec((_ST, _BG, _C), lambda bg, st: (st, bg, 0)),
        ],
        out_specs=pl.BlockSpec((_REMAIN, _BG, _C), lambda bg, st: (0, bg, 0)),
        out_shape=jax.ShapeDtypeStruct((_REMAIN, _B, _C), jnp.float32),
        compiler_params=pltpu.CompilerParams(
            dimension_semantics=("parallel", "arbitrary"),
        ),
    )(fwd_p, patches)


def kernel(patches, mask_patches):
    fwd, bwd, idx_m, fwd_p = _indexes()
    out_m = _sc_gather_mask(mask_patches.reshape(_T * _B, _C), idx_m)
    out_p = _tc_onehot_patches(patches, fwd_p)
    return (
        out_p,
        out_m.reshape(_MASKED, _B, _C),
        fwd,
        bwd,
    )


# speedup vs baseline: 3.4600x; 2.2541x over previous
import functools

import jax
import jax.numpy as jnp
from jax import lax
from jax.experimental import pallas as pl
from jax.experimental.pallas import tpu as pltpu
from jax.experimental.pallas import tpu_sc as plsc

_T, _B, _C = 576, 64, 768
_RATIO = 0.75
_REMAIN = int(_T * (1 - _RATIO))
_MASKED = _T - _REMAIN
_NC, _NS = 2, 16
_NW = _NC * _NS
_CHUNK = 32
_PC = (_REMAIN * _B) // (_NW * _CHUNK)
_MC = (_MASKED * _B) // (_NW * _CHUNK)
_DEPTH = 5
_G = 2


def _indexes():
    keys = jax.random.split(jax.random.key(42), _B)
    fwd = jax.vmap(lambda k: jax.random.permutation(k, _T))(keys).T
    bwd = jnp.argsort(fwd, axis=0)
    col = jnp.arange(_B, dtype=jnp.int32)[None, :]
    flat = fwd.astype(jnp.int32) * _B + col
    idx_p = flat[:_REMAIN].reshape(_NW, _PC, _CHUNK)
    idx_m = flat[_REMAIN:].reshape(_NW, _MC, _CHUNK)
    return fwd, bwd, idx_p, idx_m


def _sc_gather(patches_f, mask_f, idx_p, idx_m):
    mesh = plsc.VectorSubcoreMesh(core_axis_name="c", subcore_axis_name="s")

    @functools.partial(
        pl.kernel,
        mesh=mesh,
        out_type=[
            jax.ShapeDtypeStruct((_REMAIN * _B, _C), jnp.float32),
            jax.ShapeDtypeStruct((_MASKED * _B, _C), jnp.float32),
        ],
        scratch_types=(
            [
                pltpu.VMEM((_PC, _CHUNK), jnp.int32),
                pltpu.VMEM((_MC, _CHUNK), jnp.int32),
            ]
            + [pltpu.VMEM((_CHUNK, _C), jnp.float32) for _ in range(_DEPTH)]
            + [pltpu.SemaphoreType.DMA for _ in range(2 * _DEPTH)]
        ),
    )
    def k(p_hbm, m_hbm, ip_hbm, im_hbm, op_hbm, om_hbm, ipv, imv, *scratch):
        bufs = list(scratch[:_DEPTH])
        gsems = list(scratch[_DEPTH : 2 * _DEPTH])
        osems = list(scratch[2 * _DEPTH :])
        wid = lax.axis_index("s") * _NC + lax.axis_index("c")
        pltpu.sync_copy(ip_hbm.at[wid], ipv)
        pltpu.sync_copy(im_hbm.at[wid], imv)
        chunks = (
            [(p_hbm, ipv, c, op_hbm, wid * _PC + c) for c in range(_PC)]
            + [(m_hbm, imv, c, om_hbm, wid * _MC + c) for c in range(_MC)]
        )
        n = len(chunks)
        gh = [None] * n
        oh = [None] * n

        def out_copy(j):
            _, _, _, dst, blk = chunks[j]
            return pltpu.async_copy(
                bufs[j % _DEPTH],
                dst.at[pl.ds(blk * _CHUNK, _CHUNK)],
                osems[j % _DEPTH],
            )

        def gather(i):
            src, idxr, c, _, _ = chunks[i]
            return pltpu.async_copy(src.at[idxr.at[c]], bufs[i % _DEPTH], gsems[i % _DEPTH])

        for i in range(min(_G + 1, n)):
            gh[i] = gather(i)
        for i in range(n):
            gh[i].wait()
            oh[i] = out_copy(i)
            nxt = i + _G + 1
            if nxt < n:
                if nxt >= _DEPTH:
                    oh[nxt - _DEPTH].wait()
                gh[nxt] = gather(nxt)
        for j in range(max(0, n - _DEPTH), n):
            oh[j].wait()

    return k(patches_f, mask_f, idx_p, idx_m)


def kernel(patches, mask_patches):
    fwd, bwd, idx_p, idx_m = _indexes()
    out_p, out_m = _sc_gather(
        patches.reshape(_T * _B, _C),
        mask_patches.reshape(_T * _B, _C),
        idx_p,
        idx_m,
    )
    return (
        out_p.reshape(_REMAIN, _B, _C),
        out_m.reshape(_MASKED, _B, _C),
        fwd,
        bwd,
    )
